# Initial kernel scaffold; baseline (speedup 1.0000x reference)
import jax
import jax.numpy as jnp
from jax.experimental import pallas as pl


def kernel(x, edge_index, batch, W1, b1, W2, b2, Wq0, Wk0, Wv0, Wo0, bo0, Wq1, Wk1, Wv1, Wo1, bo1, Wout, bout):
    raise NotImplementedError("write your pallas kernel here")


# speedup vs baseline: 22.2990x; 22.2990x over previous
import functools

import jax
import jax.numpy as jnp
from jax import lax
from jax.experimental import pallas as pl
from jax.experimental.pallas import tpu as pltpu
from jax.experimental.pallas import tpu_sc as plsc

N = 10000
E = 320000
D_IN = 128
D_MID = 192
D = 256
H = 8
DH = 32
G = 64
INV = 1.0 / (DH ** 0.5)

NC = 2
NS = 16
NW = NC * NS
LANES = 16

RB = 400
NRB = N // RB

C2 = 40
NCH2 = E // NW // C2
C3 = 40
NCH3 = E // NS // C3
NPAD = 10240
NRS = NPAD // NS
AW = 144
ZR = 32


def _embed_body(x_ref, W1_ref, b1_ref, W2_ref, b2_ref, Wq_ref, Wk_ref, Wv_ref,
                h_ref, q_ref, k_ref, v2_ref):
    t = jnp.maximum(
        jnp.dot(x_ref[...], W1_ref[...], preferred_element_type=jnp.float32)
        + b1_ref[...], 0.0)
    h = (jnp.dot(t, W2_ref[...], preferred_element_type=jnp.float32)
         + b2_ref[...])
    h_ref[...] = h
    q_ref[...] = jnp.dot(h, Wq_ref[...], preferred_element_type=jnp.float32)
    k_ref[...] = jnp.dot(h, Wk_ref[...], preferred_element_type=jnp.float32)
    v = jnp.dot(h, Wv_ref[...], preferred_element_type=jnp.float32)
    v2_ref[...] = jnp.stack([v[:, :128], v[:, 128:]])


def _embed(x, W1, b1, W2, b2, Wq, Wk, Wv):
    full = lambda shape: pl.BlockSpec(shape, lambda i: (0,) * len(shape))
    return pl.pallas_call(
        _embed_body,
        grid=(NRB,),
        in_specs=[
            pl.BlockSpec((RB, D_IN), lambda i: (i, 0)),
            full((D_IN, D_MID)), full((1, D_MID)),
            full((D_MID, D)), full((1, D)),
            full((D, D)), full((D, D)), full((D, D)),
        ],
        out_specs=[
            pl.BlockSpec((RB, D), lambda i: (i, 0)),
            pl.BlockSpec((RB, D), lambda i: (i, 0)),
            pl.BlockSpec((RB, D), lambda i: (i, 0)),
            pl.BlockSpec((2, RB, 128), lambda i: (0, i, 0)),
        ],
        out_shape=[
            jax.ShapeDtypeStruct((N, D), jnp.float32),
            jax.ShapeDtypeStruct((N, D), jnp.float32),
            jax.ShapeDtypeStruct((N, D), jnp.float32),
            jax.ShapeDtypeStruct((2, N, 128), jnp.float32),
        ],
    )(x, W1, b1, W2, b2, Wq, Wk, Wv)


def _score_body(q_hbm, k_hbm, src_hbm, dst_hbm, sc_hbm, wmax_hbm,
                sidx0, sidx1, didx0, didx1, qbuf0, qbuf1, kbuf0, kbuf1,
                sbuf0, sbuf1, mbuf,
                ksem0, ksem1, qsem0, qsem1):
    cid = lax.axis_index("c")
    sid = lax.axis_index("s")
    wid = sid * NC + cid
    base = wid * (E // NW)
    sidxs = (sidx0, sidx1)
    didxs = (didx0, didx1)
    qbufs = (qbuf0, qbuf1)
    kbufs = (kbuf0, kbuf1)
    sbufs = (sbuf0, sbuf1)
    ksems = (ksem0, ksem1)
    qsems = (qsem0, qsem1)

    def issue(j, s):
        eb = base + j * C2
        pltpu.sync_copy(src_hbm.at[pl.ds(eb, C2)], sidxs[s])
        pltpu.sync_copy(dst_hbm.at[pl.ds(eb, C2)], didxs[s])
        pltpu.async_copy(k_hbm.at[sidxs[s]], kbufs[s], ksems[s])
        pltpu.async_copy(q_hbm.at[didxs[s]], qbufs[s], qsems[s])

    def gwait(s):
        pltpu.make_async_copy(k_hbm.at[sidxs[s]], kbufs[s], ksems[s]).wait()
        pltpu.make_async_copy(q_hbm.at[didxs[s]], qbufs[s], qsems[s]).wait()

    def compute(j, s, rmax):
        lane = lax.iota(jnp.int32, LANES)
        qbuf = qbufs[s]
        kbuf = kbufs[s]
        sbuf = sbufs[s]

        def pair(p, rm):
            svec = jnp.zeros((LANES,), jnp.float32)
            for half in range(2):
                e = 2 * p + half
                for h2 in range(H):
                    q0 = qbuf[e, pl.ds(DH * h2, LANES)]
                    q1 = qbuf[e, pl.ds(DH * h2 + LANES, LANES)]
                    k0 = kbuf[e, pl.ds(DH * h2, LANES)]
                    k1 = kbuf[e, pl.ds(DH * h2 + LANES, LANES)]
                    x = q0 * k0 + q1 * k1
                    for sh in (8, 4, 2, 1):
                        x = x + jnp.take_along_axis(x, lane ^ sh, axis=0)
                    svec = jnp.where(lane == 8 * half + h2, x, svec)
            sbuf[pl.ds(p * LANES, LANES)] = svec
            return jnp.maximum(rm, svec)
        rmax = lax.fori_loop(0, C2 // 2, pair, rmax)
        eb8 = (base + j * C2) * H
        pltpu.sync_copy(sbuf, sc_hbm.at[pl.ds(eb8, C2 * H)])
        return rmax

    issue(0, 0)
    issue(1, 1)

    def body2(u, rmax):
        j0 = 2 * u
        gwait(0)
        rmax = compute(j0, 0, rmax)

        @pl.when(j0 + 2 < NCH2)
        def _():
            issue(j0 + 2, 0)

        gwait(1)
        rmax = compute(j0 + 1, 1, rmax)

        @pl.when(j0 + 3 < NCH2)
        def _():
            issue(j0 + 3, 1)

        return rmax

    neg = jnp.full((LANES,), -jnp.inf, jnp.float32)
    rmax = lax.fori_loop(0, NCH2 // 2, body2, neg)
    mbuf[...] = rmax
    pltpu.sync_copy(mbuf, wmax_hbm.at[pl.ds(wid * LANES, LANES)])


def _score(q, k, src, dst):
    mesh = plsc.VectorSubcoreMesh(core_axis_name="c", subcore_axis_name="s")
    f = pl.kernel(
        _score_body,
        mesh=mesh,
        out_type=(
            jax.ShapeDtypeStruct((E * H,), jnp.float32),
            jax.ShapeDtypeStruct((NW * LANES,), jnp.float32),
        ),
        scratch_types=[
            pltpu.VMEM((C2,), jnp.int32),
            pltpu.VMEM((C2,), jnp.int32),
            pltpu.VMEM((C2,), jnp.int32),
            pltpu.VMEM((C2,), jnp.int32),
            pltpu.VMEM((C2, D), jnp.float32),
            pltpu.VMEM((C2, D), jnp.float32),
            pltpu.VMEM((C2, D), jnp.float32),
            pltpu.VMEM((C2, D), jnp.float32),
            pltpu.VMEM((C2 * H,), jnp.float32),
            pltpu.VMEM((C2 * H,), jnp.float32),
            pltpu.VMEM((LANES,), jnp.float32),
            pltpu.SemaphoreType.DMA,
            pltpu.SemaphoreType.DMA,
            pltpu.SemaphoreType.DMA,
            pltpu.SemaphoreType.DMA,
        ],
    )
    return f(q, k, src, dst)


def _scatter_body(sc_hbm, wmax_hbm, v2_hbm, src_hbm, dst_hbm,
                  nm_hbm, dn_hbm,
                  srcidx0, srcidx1, dstidx0, dstidx1, dstb0, dstb1,
                  vbuf0, vbuf1, sbuf0, sbuf1, msgbuf0, msgbuf1, wmbuf,
                  numer_sh,
                  gsem0, gsem1, ssem0, ssem1):
    cid = lax.axis_index("c")
    sid = lax.axis_index("s")
    base = sid * (E // NS)
    srcidxs = (srcidx0, srcidx1)
    dstidxs = (dstidx0, dstidx1)
    dstbs = (dstb0, dstb1)
    vbufs = (vbuf0, vbuf1)
    sbufs = (sbuf0, sbuf1)
    msgbufs = (msgbuf0, msgbuf1)
    gsems = (gsem0, gsem1)
    ssems = (ssem0, ssem1)

    pltpu.sync_copy(wmax_hbm, wmbuf)
    m = wmbuf[pl.ds(0, LANES)]
    for i in range(1, NW):
        m = jnp.maximum(m, wmbuf[pl.ds(i * LANES, LANES)])
    swap = lax.iota(jnp.int32, LANES) ^ 8
    m16 = jnp.maximum(m, jnp.take_along_axis(m, swap, axis=0))

    zero16 = jnp.zeros((LANES,), jnp.float32)

    def zrow(i, carry):
        for t in range(AW // LANES):
            msgbuf0[i, pl.ds(LANES * t, LANES)] = zero16
        return carry
    lax.fori_loop(0, ZR, zrow, 0)
    for t in range(NRS // ZR):
        pltpu.sync_copy(msgbuf0.at[pl.ds(0, ZR), :],
                        numer_sh.at[pl.ds(sid * NRS + t * ZR, ZR), :])
    sbuf0[pl.ds(C3 * H, LANES)] = zero16
    sbuf1[pl.ds(C3 * H, LANES)] = zero16
    plsc.subcore_barrier()

    def issue3(j, s):
        eb = base + j * C3
        pltpu.sync_copy(src_hbm.at[pl.ds(eb, C3)], srcidxs[s])
        pltpu.sync_copy(dst_hbm.at[pl.ds(eb, C3)], dstidxs[s])
        pltpu.sync_copy(sc_hbm.at[pl.ds(eb * H, C3 * H)],
                        sbufs[s].at[pl.ds(0, C3 * H)])
        pltpu.async_copy(v2_hbm.at[cid].at[srcidxs[s]], vbufs[s], gsems[s])

    def gwait3(s):
        pltpu.make_async_copy(
            v2_hbm.at[cid].at[srcidxs[s]], vbufs[s], gsems[s]).wait()

    def swait3(s):
        pltpu.make_async_copy(
            msgbufs[s], numer_sh.at[dstbs[s]], ssems[s]).wait()

    def compute3(j, s):
        for t in (0, LANES, C3 - LANES):
            dstbs[s][pl.ds(t, LANES)] = dstidxs[s][pl.ds(t, LANES)]
        hbase = 4 * cid
        sbuf = sbufs[s]
        vbuf = vbufs[s]
        msgbuf = msgbufs[s]

        def edge(e, carry):
            sv = sbuf[pl.ds(e * H, LANES)]
            exv = jnp.exp((sv - m16) * INV)
            msgbuf[e, pl.ds(128, LANES)] = exv
            for hh in range(4):
                gidx = jnp.zeros((LANES,), jnp.int32) + (hbase + hh)
                exh = jnp.take_along_axis(exv, gidx, axis=0)
                v0 = vbuf[e, pl.ds(DH * hh, LANES)]
                v1 = vbuf[e, pl.ds(DH * hh + LANES, LANES)]
                msgbuf[e, pl.ds(DH * hh, LANES)] = exh * v0
                msgbuf[e, pl.ds(DH * hh + LANES, LANES)] = exh * v1
            return carry
        lax.fori_loop(0, C3, edge, 0, unroll=2)
        pltpu.async_copy(msgbuf, numer_sh.at[dstbs[s]], ssems[s], add=True)

    issue3(0, 0)
    issue3(1, 1)

    def body3(u, carry):
        for s in range(2):
            j = 2 * u + s
            gwait3(s)

            @pl.when(u >= 1)
            def _():
                swait3(s)

            compute3(j, s)

            @pl.when(j + 2 < NCH3)
            def _():
                issue3(j + 2, s)
        return carry
    lax.fori_loop(0, NCH3 // 2, body3, 0)
    swait3(0)
    swait3(1)
    plsc.subcore_barrier()

    rb = sid * NRS
    pltpu.sync_copy(numer_sh.at[pl.ds(rb, NRS), pl.ds(0, 128)],
                    nm_hbm.at[cid].at[pl.ds(rb, NRS)])
    pltpu.sync_copy(numer_sh.at[pl.ds(rb, NRS), pl.ds(128, 8)],
                    dn_hbm.at[cid].at[pl.ds(rb, NRS)])


def _scatter(sc, wmax, v2, src, dst):
    mesh = plsc.VectorSubcoreMesh(core_axis_name="c", subcore_axis_name="s")
    f = pl.kernel(
        _scatter_body,
        mesh=mesh,
        compiler_params=pltpu.CompilerParams(use_tc_tiling_on_sc=False),
        out_type=(
            jax.ShapeDtypeStruct((2, NPAD, 128), jnp.float32),
            jax.ShapeDtypeStruct((2, NPAD, 8), jnp.float32),
        ),
        scratch_types=[
            pltpu.VMEM((C3,), jnp.int32),
            pltpu.VMEM((C3,), jnp.int32),
            pltpu.VMEM((C3,), jnp.int32),
            pltpu.VMEM((C3,), jnp.int32),
            pltpu.VMEM((C3,), jnp.int32),
            pltpu.VMEM((C3,), jnp.int32),
            pltpu.VMEM((C3, 128), jnp.float32),
            pltpu.VMEM((C3, 128), jnp.float32),
            pltpu.VMEM((C3 * H + LANES,), jnp.float32),
            pltpu.VMEM((C3 * H + LANES,), jnp.float32),
            pltpu.VMEM((C3, AW), jnp.float32),
            pltpu.VMEM((C3, AW), jnp.float32),
            pltpu.VMEM((NW * LANES,), jnp.float32),
            pltpu.VMEM_SHARED((NPAD, AW), jnp.float32),
            pltpu.SemaphoreType.DMA,
            pltpu.SemaphoreType.DMA,
            pltpu.SemaphoreType.DMA,
            pltpu.SemaphoreType.DMA,
        ],
    )
    return f(sc, wmax, v2, src, dst)


def _attn_out(h_ref, nm_ref, dn_ref, Wo_ref, bo_ref):
    nm = nm_ref[...]
    agg = jnp.concatenate([nm[0], nm[1]], axis=1)
    den = dn_ref[...][0]
    ei = (lax.broadcasted_iota(jnp.int32, (H, D), 1) // DH
          == lax.broadcasted_iota(jnp.int32, (H, D), 0)).astype(jnp.float32)
    den256 = jnp.dot(den, ei, preferred_element_type=jnp.float32)
    aggn = agg / (den256 + 1e-16)
    return jnp.maximum(
        h_ref[...]
        + jnp.dot(aggn, Wo_ref[...], preferred_element_type=jnp.float32)
        + bo_ref[...], 0.0)


def _epi_body(h_ref, nm_ref, dn_ref, Wo_ref, bo_ref, Wq_ref, Wk_ref, Wv_ref,
              h2_ref, q_ref, k_ref, v2_ref):
    h2 = _attn_out(h_ref, nm_ref, dn_ref, Wo_ref, bo_ref)
    h2_ref[...] = h2
    q_ref[...] = jnp.dot(h2, Wq_ref[...], preferred_element_type=jnp.float32)
    k_ref[...] = jnp.dot(h2, Wk_ref[...], preferred_element_type=jnp.float32)
    v = jnp.dot(h2, Wv_ref[...], preferred_element_type=jnp.float32)
    v2_ref[...] = jnp.stack([v[:, :128], v[:, 128:]])


def _epi(h, nm, dn, Wo, bo, Wq, Wk, Wv):
    full = lambda shape: pl.BlockSpec(shape, lambda i: (0,) * len(shape))
    return pl.pallas_call(
        _epi_body,
        grid=(NRB,),
        in_specs=[
            pl.BlockSpec((RB, D), lambda i: (i, 0)),
            pl.BlockSpec((2, RB, 128), lambda i: (0, i, 0)),
            pl.BlockSpec((2, RB, 8), lambda i: (0, i, 0)),
            full((D, D)), full((1, D)),
            full((D, D)), full((D, D)), full((D, D)),
        ],
        out_specs=[
            pl.BlockSpec((RB, D), lambda i: (i, 0)),
            pl.BlockSpec((RB, D), lambda i: (i, 0)),
            pl.BlockSpec((RB, D), lambda i: (i, 0)),
            pl.BlockSpec((2, RB, 128), lambda i: (0, i, 0)),
        ],
        out_shape=[
            jax.ShapeDtypeStruct((N, D), jnp.float32),
            jax.ShapeDtypeStruct((N, D), jnp.float32),
            jax.ShapeDtypeStruct((N, D), jnp.float32),
            jax.ShapeDtypeStruct((2, N, 128), jnp.float32),
        ],
    )(h, nm, dn, Wo, bo, Wq, Wk, Wv)


def _final_body(h_ref, nm_ref, dn_ref, Wo_ref, bo_ref, batch_ref, WoutT_ref,
                bout_ref, out_ref, sums_ref, cnt_ref):
    i = pl.program_id(0)

    @pl.when(i == 0)
    def _():
        sums_ref[...] = jnp.zeros_like(sums_ref)
        cnt_ref[...] = jnp.zeros_like(cnt_ref)

    h2 = _attn_out(h_ref, nm_ref, dn_ref, Wo_ref, bo_ref)
    b = batch_ref[0, 0, :]
    onehot = (b[:, None]
              == lax.broadcasted_iota(jnp.int32, (RB, G), 1)).astype(jnp.float32)
    sums_ref[...] += lax.dot_general(
        onehot, h2, (((0,), (0,)), ((), ())),
        preferred_element_type=jnp.float32)
    cnt_ref[...] += jnp.sum(onehot, axis=0)[:, None]

    @pl.when(i == NRB - 1)
    def _():
        gf = sums_ref[...] / jnp.maximum(cnt_ref[...], 1.0)
        out_ref[...] = (jnp.sum(gf * WoutT_ref[...], axis=1, keepdims=True)
                        + bout_ref[...])


def _final(h, nm, dn, Wo, bo, batch3, WoutT, bout):
    full = lambda shape: pl.BlockSpec(shape, lambda i: (0,) * len(shape))
    return pl.pallas_call(
        _final_body,
        grid=(NRB,),
        in_specs=[
            pl.BlockSpec((RB, D), lambda i: (i, 0)),
            pl.BlockSpec((2, RB, 128), lambda i: (0, i, 0)),
            pl.BlockSpec((2, RB, 8), lambda i: (0, i, 0)),
            full((D, D)), full((1, D)),
            pl.BlockSpec((1, 1, RB), lambda i: (i, 0, 0)),
            full((1, D)), full((1, 1)),
        ],
        out_specs=pl.BlockSpec((G, 1), lambda i: (0, 0)),
        out_shape=jax.ShapeDtypeStruct((G, 1), jnp.float32),
        scratch_shapes=[
            pltpu.VMEM((G, D), jnp.float32),
            pltpu.VMEM((G, D), jnp.float32),
        ],
    )(h, nm, dn, Wo, bo, batch3, WoutT, bout)


def kernel(x, edge_index, batch, W1, b1, W2, b2, Wq0, Wk0, Wv0, Wo0, bo0,
           Wq1, Wk1, Wv1, Wo1, bo1, Wout, bout):
    src = edge_index[0]
    dst = edge_index[1]
    h0, q0, k0, v20 = _embed(x, W1, b1.reshape(1, -1), W2, b2.reshape(1, -1),
                             Wq0, Wk0, Wv0)
    sc0, wm0 = _score(q0, k0, src, dst)
    nm0, dn0 = _scatter(sc0, wm0, v20, src, dst)
    h1, q1, k1, v21 = _epi(h0, nm0, dn0, Wo0, bo0.reshape(1, -1),
                           Wq1, Wk1, Wv1)
    sc1, wm1 = _score(q1, k1, src, dst)
    nm1, dn1 = _scatter(sc1, wm1, v21, src, dst)
    out = _final(h1, nm1, dn1, Wo1, bo1.reshape(1, -1),
                 batch.reshape(NRB, 1, RB), Wout.reshape(1, -1),
                 bout.reshape(1, 1))
    return out


# speedup vs baseline: 39.7861x; 1.7842x over previous
import functools

import jax
import jax.numpy as jnp
from jax import lax
from jax.experimental import pallas as pl
from jax.experimental.pallas import tpu as pltpu
from jax.experimental.pallas import tpu_sc as plsc

N = 10000
E = 320000
D_IN = 128
D_MID = 192
D = 256
H = 8
DH = 32
G = 64
INV = 1.0 / (DH ** 0.5)

NC = 2
NS = 16
NW = NC * NS
LANES = 16

RB = 400
NRB = N // RB

C2 = 40
NCH2 = E // NW // C2
C3 = 40
NCH3 = E // NS // C3
NPAD = 10240
NRS = NPAD // NS
AW = 144
ZR = 32


def _embed_body(x_ref, W1_ref, b1_ref, W2_ref, b2_ref, Wq_ref, Wk_ref, Wv_ref,
                h_ref, q_ref, k_ref, v2_ref):
    t = jnp.maximum(
        jnp.dot(x_ref[...], W1_ref[...], preferred_element_type=jnp.float32)
        + b1_ref[...], 0.0)
    h = (jnp.dot(t, W2_ref[...], preferred_element_type=jnp.float32)
         + b2_ref[...])
    h_ref[...] = h
    q_ref[...] = jnp.dot(h, Wq_ref[...], preferred_element_type=jnp.float32)
    k_ref[...] = jnp.dot(h, Wk_ref[...], preferred_element_type=jnp.float32)
    v = jnp.dot(h, Wv_ref[...], preferred_element_type=jnp.float32)
    v2_ref[...] = jnp.stack([v[:, :128], v[:, 128:]])


def _embed(x, W1, b1, W2, b2, Wq, Wk, Wv):
    full = lambda shape: pl.BlockSpec(shape, lambda i: (0,) * len(shape))
    return pl.pallas_call(
        _embed_body,
        grid=(NRB,),
        in_specs=[
            pl.BlockSpec((RB, D_IN), lambda i: (i, 0)),
            full((D_IN, D_MID)), full((1, D_MID)),
            full((D_MID, D)), full((1, D)),
            full((D, D)), full((D, D)), full((D, D)),
        ],
        out_specs=[
            pl.BlockSpec((RB, D), lambda i: (i, 0)),
            pl.BlockSpec((RB, D), lambda i: (i, 0)),
            pl.BlockSpec((RB, D), lambda i: (i, 0)),
            pl.BlockSpec((2, RB, 128), lambda i: (0, i, 0)),
        ],
        out_shape=[
            jax.ShapeDtypeStruct((N, D), jnp.float32),
            jax.ShapeDtypeStruct((N, D), jnp.float32),
            jax.ShapeDtypeStruct((N, D), jnp.float32),
            jax.ShapeDtypeStruct((2, N, 128), jnp.float32),
        ],
    )(x, W1, b1, W2, b2, Wq, Wk, Wv)


def _score_body(q_hbm, k_hbm, src_hbm, dst_hbm, sc_hbm, wmax_hbm,
                sidx_all, didx_all, qbuf0, qbuf1, kbuf0, kbuf1,
                sbuf0, sbuf1, mbuf,
                ksem0, ksem1, qsem0, qsem1, osem0, osem1):
    cid = lax.axis_index("c")
    sid = lax.axis_index("s")
    wid = sid * NC + cid
    base = wid * (E // NW)
    qbufs = (qbuf0, qbuf1)
    kbufs = (kbuf0, kbuf1)
    sbufs = (sbuf0, sbuf1)
    ksems = (ksem0, ksem1)
    qsems = (qsem0, qsem1)
    osems = (osem0, osem1)

    pltpu.sync_copy(src_hbm.at[pl.ds(base, E // NW)], sidx_all)
    pltpu.sync_copy(dst_hbm.at[pl.ds(base, E // NW)], didx_all)

    def issue(j, s):
        pltpu.async_copy(k_hbm.at[sidx_all.at[pl.ds(j * C2, C2)]],
                         kbufs[s], ksems[s])
        pltpu.async_copy(q_hbm.at[didx_all.at[pl.ds(j * C2, C2)]],
                         qbufs[s], qsems[s])

    def gwait(j, s):
        pltpu.make_async_copy(k_hbm.at[sidx_all.at[pl.ds(j * C2, C2)]],
                              kbufs[s], ksems[s]).wait()
        pltpu.make_async_copy(q_hbm.at[didx_all.at[pl.ds(j * C2, C2)]],
                              qbufs[s], qsems[s]).wait()

    def owait(j, s):
        eb8 = (base + j * C2) * H
        pltpu.make_async_copy(sbufs[s],
                              sc_hbm.at[pl.ds(eb8, C2 * H)], osems[s]).wait()

    def compute(j, s, rmax):
        lane = lax.iota(jnp.int32, LANES)
        qbuf = qbufs[s]
        kbuf = kbufs[s]
        sbuf = sbufs[s]

        def pair(p, rm):
            svec = jnp.zeros((LANES,), jnp.float32)
            for half in range(2):
                e = 2 * p + half
                for h2 in range(H):
                    q0 = qbuf[e, pl.ds(DH * h2, LANES)]
                    q1 = qbuf[e, pl.ds(DH * h2 + LANES, LANES)]
                    k0 = kbuf[e, pl.ds(DH * h2, LANES)]
                    k1 = kbuf[e, pl.ds(DH * h2 + LANES, LANES)]
                    x = q0 * k0 + q1 * k1
                    for sh in (8, 4, 2, 1):
                        x = x + jnp.take_along_axis(x, lane ^ sh, axis=0)
                    svec = jnp.where(lane == 8 * half + h2, x, svec)
            sbuf[pl.ds(p * LANES, LANES)] = svec
            return jnp.maximum(rm, svec)
        rmax = lax.fori_loop(0, C2 // 2, pair, rmax)
        eb8 = (base + j * C2) * H
        pltpu.async_copy(sbuf, sc_hbm.at[pl.ds(eb8, C2 * H)], osems[s])
        return rmax

    issue(0, 0)
    issue(1, 1)

    def body2(u, rmax):
        j0 = 2 * u
        gwait(j0, 0)

        @pl.when(u >= 1)
        def _():
            owait(j0 - 2, 0)

        rmax = compute(j0, 0, rmax)

        @pl.when(j0 + 2 < NCH2)
        def _():
            issue(j0 + 2, 0)

        gwait(j0 + 1, 1)

        @pl.when(u >= 1)
        def _():
            owait(j0 - 1, 1)

        rmax = compute(j0 + 1, 1, rmax)

        @pl.when(j0 + 3 < NCH2)
        def _():
            issue(j0 + 3, 1)

        return rmax

    neg = jnp.full((LANES,), -jnp.inf, jnp.float32)
    rmax = lax.fori_loop(0, NCH2 // 2, body2, neg)
    owait(NCH2 - 2, 0)
    owait(NCH2 - 1, 1)
    mbuf[...] = rmax
    pltpu.sync_copy(mbuf, wmax_hbm.at[pl.ds(wid * LANES, LANES)])


def _score(q, k, src, dst):
    mesh = plsc.VectorSubcoreMesh(core_axis_name="c", subcore_axis_name="s")
    f = pl.kernel(
        _score_body,
        mesh=mesh,
        out_type=(
            jax.ShapeDtypeStruct((E * H,), jnp.float32),
            jax.ShapeDtypeStruct((NW * LANES,), jnp.float32),
        ),
        scratch_types=[
            pltpu.VMEM((E // NW,), jnp.int32),
            pltpu.VMEM((E // NW,), jnp.int32),
            pltpu.VMEM((C2, D), jnp.float32),
            pltpu.VMEM((C2, D), jnp.float32),
            pltpu.VMEM((C2, D), jnp.float32),
            pltpu.VMEM((C2, D), jnp.float32),
            pltpu.VMEM((C2 * H,), jnp.float32),
            pltpu.VMEM((C2 * H,), jnp.float32),
            pltpu.VMEM((LANES,), jnp.float32),
            pltpu.SemaphoreType.DMA,
            pltpu.SemaphoreType.DMA,
            pltpu.SemaphoreType.DMA,
            pltpu.SemaphoreType.DMA,
            pltpu.SemaphoreType.DMA,
            pltpu.SemaphoreType.DMA,
        ],
    )
    return f(q, k, src, dst)


def _scatter_body(sc_hbm, wmax_hbm, v2_hbm, src_hbm, dst_hbm,
                  nm_hbm, dn_hbm,
                  srcidx0, srcidx1, srcidx2, srcidx3,
                  dstidx0, dstidx1, dstidx2, dstidx3,
                  sbuf0, sbuf1, sbuf2, sbuf3,
                  dstb0, dstb1, vbuf0, vbuf1, msgbuf0, msgbuf1, wmbuf,
                  numer_sh,
                  msem0, msem1, msem2, msem3, gsem0, gsem1, ssem0, ssem1):
    cid = lax.axis_index("c")
    sid = lax.axis_index("s")
    base = sid * (E // NS)
    srcidxs = (srcidx0, srcidx1, srcidx2, srcidx3)
    dstidxs = (dstidx0, dstidx1, dstidx2, dstidx3)
    sbufs = (sbuf0, sbuf1, sbuf2, sbuf3)
    dstbs = (dstb0, dstb1)
    vbufs = (vbuf0, vbuf1)
    msgbufs = (msgbuf0, msgbuf1)
    msems = (msem0, msem1, msem2, msem3)
    gsems = (gsem0, gsem1)
    ssems = (ssem0, ssem1)

    pltpu.sync_copy(wmax_hbm, wmbuf)
    m = wmbuf[pl.ds(0, LANES)]
    for i in range(1, NW):
        m = jnp.maximum(m, wmbuf[pl.ds(i * LANES, LANES)])
    swap = lax.iota(jnp.int32, LANES) ^ 8
    m16 = jnp.maximum(m, jnp.take_along_axis(m, swap, axis=0))

    zero16 = jnp.zeros((LANES,), jnp.float32)

    def zrow(i, carry):
        for t in range(AW // LANES):
            msgbuf0[i, pl.ds(LANES * t, LANES)] = zero16
        return carry
    lax.fori_loop(0, ZR, zrow, 0)
    for t in range(NRS // ZR):
        pltpu.sync_copy(msgbuf0.at[pl.ds(0, ZR), :],
                        numer_sh.at[pl.ds(sid * NRS + t * ZR, ZR), :])
    for sb in sbufs:
        sb[pl.ds(C3 * H, LANES)] = zero16
    plsc.subcore_barrier()

    def smalls(j, a):
        eb = base + j * C3
        pltpu.async_copy(src_hbm.at[pl.ds(eb, C3)], srcidxs[a], msems[a])
        pltpu.async_copy(dst_hbm.at[pl.ds(eb, C3)], dstidxs[a], msems[a])
        pltpu.async_copy(sc_hbm.at[pl.ds(eb * H, C3 * H)],
                         sbufs[a].at[pl.ds(0, C3 * H)], msems[a])

    def mwait(j, a):
        eb = base + j * C3
        pltpu.make_async_copy(src_hbm.at[pl.ds(eb, C3)], srcidxs[a],
                              msems[a]).wait()
        pltpu.make_async_copy(dst_hbm.at[pl.ds(eb, C3)], dstidxs[a],
                              msems[a]).wait()
        pltpu.make_async_copy(sc_hbm.at[pl.ds(eb * H, C3 * H)],
                              sbufs[a].at[pl.ds(0, C3 * H)], msems[a]).wait()

    def gath(j, a, g):
        mwait(j, a)
        pltpu.async_copy(v2_hbm.at[cid].at[srcidxs[a]], vbufs[g], gsems[g])

    def gwait3(a, g):
        pltpu.make_async_copy(
            v2_hbm.at[cid].at[srcidxs[a]], vbufs[g], gsems[g]).wait()

    def swait3(g):
        pltpu.make_async_copy(
            msgbufs[g], numer_sh.at[dstbs[g]], ssems[g]).wait()

    def compute3(j, a, g):
        for t in (0, LANES, C3 - LANES):
            dstbs[g][pl.ds(t, LANES)] = dstidxs[a][pl.ds(t, LANES)]
        hbase = 4 * cid
        sbuf = sbufs[a]
        vbuf = vbufs[g]
        msgbuf = msgbufs[g]

        def edge(e, carry):
            sv = sbuf[pl.ds(e * H, LANES)]
            exv = jnp.exp((sv - m16) * INV)
            msgbuf[e, pl.ds(128, LANES)] = exv
            for hh in range(4):
                gidx = jnp.zeros((LANES,), jnp.int32) + (hbase + hh)
                exh = jnp.take_along_axis(exv, gidx, axis=0)
                v0 = vbuf[e, pl.ds(DH * hh, LANES)]
                v1 = vbuf[e, pl.ds(DH * hh + LANES, LANES)]
                msgbuf[e, pl.ds(DH * hh, LANES)] = exh * v0
                msgbuf[e, pl.ds(DH * hh + LANES, LANES)] = exh * v1
            return carry
        lax.fori_loop(0, C3, edge, 0, unroll=2)
        pltpu.async_copy(msgbuf, numer_sh.at[dstbs[g]], ssems[g], add=True)

    for a in range(4):
        smalls(a, a)
    gath(0, 0, 0)
    gath(1, 1, 1)

    def body3(u, carry):
        for k in range(4):
            j = 4 * u + k
            a = k
            g = k % 2
            gwait3(a, g)
            if k >= 2:
                swait3(g)
            else:
                @pl.when(u >= 1)
                def _():
                    swait3(g)
            compute3(j, a, g)

            @pl.when(u < NCH3 // 4 - 1)
            def _():
                smalls(j + 4, a)
            if k < 2:
                gath2 = lambda: gath(j + 2, (a + 2) % 4, g)
                gath2()
            else:
                @pl.when(u < NCH3 // 4 - 1)
                def _():
                    gath(j + 2, (a + 2) % 4, g)
        return carry
    lax.fori_loop(0, NCH3 // 4, body3, 0)
    swait3(0)
    swait3(1)
    plsc.subcore_barrier()

    rb = sid * NRS
    pltpu.sync_copy(numer_sh.at[pl.ds(rb, NRS), pl.ds(0, 128)],
                    nm_hbm.at[cid].at[pl.ds(rb, NRS)])
    pltpu.sync_copy(numer_sh.at[pl.ds(rb, NRS), pl.ds(128, 8)],
                    dn_hbm.at[cid].at[pl.ds(rb, NRS)])


def _scatter(sc, wmax, v2, src, dst):
    mesh = plsc.VectorSubcoreMesh(core_axis_name="c", subcore_axis_name="s")
    f = pl.kernel(
        _scatter_body,
        mesh=mesh,
        compiler_params=pltpu.CompilerParams(use_tc_tiling_on_sc=False),
        out_type=(
            jax.ShapeDtypeStruct((2, NPAD, 128), jnp.float32),
            jax.ShapeDtypeStruct((2, NPAD, 8), jnp.float32),
        ),
        scratch_types=[
            pltpu.VMEM((C3,), jnp.int32),
            pltpu.VMEM((C3,), jnp.int32),
            pltpu.VMEM((C3,), jnp.int32),
            pltpu.VMEM((C3,), jnp.int32),
            pltpu.VMEM((C3,), jnp.int32),
            pltpu.VMEM((C3,), jnp.int32),
            pltpu.VMEM((C3,), jnp.int32),
            pltpu.VMEM((C3,), jnp.int32),
            pltpu.VMEM((C3 * H + LANES,), jnp.float32),
            pltpu.VMEM((C3 * H + LANES,), jnp.float32),
            pltpu.VMEM((C3 * H + LANES,), jnp.float32),
            pltpu.VMEM((C3 * H + LANES,), jnp.float32),
            pltpu.VMEM((C3,), jnp.int32),
            pltpu.VMEM((C3,), jnp.int32),
            pltpu.VMEM((C3, 128), jnp.float32),
            pltpu.VMEM((C3, 128), jnp.float32),
            pltpu.VMEM((C3, AW), jnp.float32),
            pltpu.VMEM((C3, AW), jnp.float32),
            pltpu.VMEM((NW * LANES,), jnp.float32),
            pltpu.VMEM_SHARED((NPAD, AW), jnp.float32),
            pltpu.SemaphoreType.DMA,
            pltpu.SemaphoreType.DMA,
            pltpu.SemaphoreType.DMA,
            pltpu.SemaphoreType.DMA,
            pltpu.SemaphoreType.DMA,
            pltpu.SemaphoreType.DMA,
            pltpu.SemaphoreType.DMA,
            pltpu.SemaphoreType.DMA,
        ],
    )
    return f(sc, wmax, v2, src, dst)


def _attn_out(h_ref, nm_ref, dn_ref, Wo_ref, bo_ref):
    nm = nm_ref[...]
    agg = jnp.concatenate([nm[0], nm[1]], axis=1)
    den = dn_ref[...][0]
    ei = (lax.broadcasted_iota(jnp.int32, (H, D), 1) // DH
          == lax.broadcasted_iota(jnp.int32, (H, D), 0)).astype(jnp.float32)
    den256 = jnp.dot(den, ei, preferred_element_type=jnp.float32)
    aggn = agg / (den256 + 1e-16)
    return jnp.maximum(
        h_ref[...]
        + jnp.dot(aggn, Wo_ref[...], preferred_element_type=jnp.float32)
        + bo_ref[...], 0.0)


def _epi_body(h_ref, nm_ref, dn_ref, Wo_ref, bo_ref, Wq_ref, Wk_ref, Wv_ref,
              h2_ref, q_ref, k_ref, v2_ref):
    h2 = _attn_out(h_ref, nm_ref, dn_ref, Wo_ref, bo_ref)
    h2_ref[...] = h2
    q_ref[...] = jnp.dot(h2, Wq_ref[...], preferred_element_type=jnp.float32)
    k_ref[...] = jnp.dot(h2, Wk_ref[...], preferred_element_type=jnp.float32)
    v = jnp.dot(h2, Wv_ref[...], preferred_element_type=jnp.float32)
    v2_ref[...] = jnp.stack([v[:, :128], v[:, 128:]])


def _epi(h, nm, dn, Wo, bo, Wq, Wk, Wv):
    full = lambda shape: pl.BlockSpec(shape, lambda i: (0,) * len(shape))
    return pl.pallas_call(
        _epi_body,
        grid=(NRB,),
        in_specs=[
            pl.BlockSpec((RB, D), lambda i: (i, 0)),
            pl.BlockSpec((2, RB, 128), lambda i: (0, i, 0)),
            pl.BlockSpec((2, RB, 8), lambda i: (0, i, 0)),
            full((D, D)), full((1, D)),
            full((D, D)), full((D, D)), full((D, D)),
        ],
        out_specs=[
            pl.BlockSpec((RB, D), lambda i: (i, 0)),
            pl.BlockSpec((RB, D), lambda i: (i, 0)),
            pl.BlockSpec((RB, D), lambda i: (i, 0)),
            pl.BlockSpec((2, RB, 128), lambda i: (0, i, 0)),
        ],
        out_shape=[
            jax.ShapeDtypeStruct((N, D), jnp.float32),
            jax.ShapeDtypeStruct((N, D), jnp.float32),
            jax.ShapeDtypeStruct((N, D), jnp.float32),
            jax.ShapeDtypeStruct((2, N, 128), jnp.float32),
        ],
    )(h, nm, dn, Wo, bo, Wq, Wk, Wv)


def _final_body(h_ref, nm_ref, dn_ref, Wo_ref, bo_ref, batch_ref, WoutT_ref,
                bout_ref, out_ref, sums_ref, cnt_ref):
    i = pl.program_id(0)

    @pl.when(i == 0)
    def _():
        sums_ref[...] = jnp.zeros_like(sums_ref)
        cnt_ref[...] = jnp.zeros_like(cnt_ref)

    h2 = _attn_out(h_ref, nm_ref, dn_ref, Wo_ref, bo_ref)
    b = batch_ref[0, 0, :]
    onehot = (b[:, None]
              == lax.broadcasted_iota(jnp.int32, (RB, G), 1)).astype(jnp.float32)
    sums_ref[...] += lax.dot_general(
        onehot, h2, (((0,), (0,)), ((), ())),
        preferred_element_type=jnp.float32)
    cnt_ref[...] += jnp.sum(onehot, axis=0)[:, None]

    @pl.when(i == NRB - 1)
    def _():
        gf = sums_ref[...] / jnp.maximum(cnt_ref[...], 1.0)
        out_ref[...] = (jnp.sum(gf * WoutT_ref[...], axis=1, keepdims=True)
                        + bout_ref[...])


def _final(h, nm, dn, Wo, bo, batch3, WoutT, bout):
    full = lambda shape: pl.BlockSpec(shape, lambda i: (0,) * len(shape))
    return pl.pallas_call(
        _final_body,
        grid=(NRB,),
        in_specs=[
            pl.BlockSpec((RB, D), lambda i: (i, 0)),
            pl.BlockSpec((2, RB, 128), lambda i: (0, i, 0)),
            pl.BlockSpec((2, RB, 8), lambda i: (0, i, 0)),
            full((D, D)), full((1, D)),
            pl.BlockSpec((1, 1, RB), lambda i: (i, 0, 0)),
            full((1, D)), full((1, 1)),
        ],
        out_specs=pl.BlockSpec((G, 1), lambda i: (0, 0)),
        out_shape=jax.ShapeDtypeStruct((G, 1), jnp.float32),
        scratch_shapes=[
            pltpu.VMEM((G, D), jnp.float32),
            pltpu.VMEM((G, D), jnp.float32),
        ],
    )(h, nm, dn, Wo, bo, batch3, WoutT, bout)


def kernel(x, edge_index, batch, W1, b1, W2, b2, Wq0, Wk0, Wv0, Wo0, bo0,
           Wq1, Wk1, Wv1, Wo1, bo1, Wout, bout):
    src = edge_index[0]
    dst = edge_index[1]
    h0, q0, k0, v20 = _embed(x, W1, b1.reshape(1, -1), W2, b2.reshape(1, -1),
                             Wq0, Wk0, Wv0)
    sc0, wm0 = _score(q0, k0, src, dst)
    nm0, dn0 = _scatter(sc0, wm0, v20, src, dst)
    h1, q1, k1, v21 = _epi(h0, nm0, dn0, Wo0, bo0.reshape(1, -1),
                           Wq1, Wk1, Wv1)
    sc1, wm1 = _score(q1, k1, src, dst)
    nm1, dn1 = _scatter(sc1, wm1, v21, src, dst)
    out = _final(h1, nm1, dn1, Wo1, bo1.reshape(1, -1),
                 batch.reshape(NRB, 1, RB), Wout.reshape(1, -1),
                 bout.reshape(1, 1))
    return out
